# baseline (speedup 1.0000x reference)
import jax
import jax.numpy as jnp
from jax.experimental import pallas as pl
from jax.experimental.pallas import tpu as pltpu

_B = 4096
_NC = 100
_HB = 64
_M = 16.0
_ALPHA = 0.1
_BETA = 0.1

_STEPS = 8
_CB = _B // _STEPS


def _loss_kernel(hash_ref, cls_ref, target_ref, tv_ref, out_ref, acc_ref):
    i = pl.program_id(0)

    @pl.when(i == 0)
    def _init():
        for k in range(6):
            acc_ref[k] = 0.0

    tgt = target_ref[...]
    col = jax.lax.broadcasted_iota(jnp.int32, (_CB, _NC), 1)
    row_max = jnp.max(tgt, axis=1, keepdims=True)
    label = jnp.min(jnp.where(tgt == row_max, col, _NC), axis=1, keepdims=True)
    onehot = (col == label).astype(jnp.float32)

    def ce_sum(logits):
        m = jnp.max(logits, axis=1, keepdims=True)
        lse = m[:, 0] + jnp.log(jnp.sum(jnp.exp(logits - m), axis=1))
        picked = jnp.sum(onehot * logits, axis=1)
        return jnp.sum(lse - picked)

    ce0 = ce_sum(cls_ref[0])
    ce1 = ce_sum(cls_ref[1])

    t = jnp.dot(onehot.astype(jnp.bfloat16), tv_ref[...].astype(jnp.bfloat16),
                preferred_element_type=jnp.float32)

    h0 = hash_ref[0]
    h1 = hash_ref[1]
    pol0 = jnp.sum(jnp.maximum(_M - h0 * t, 0.0))
    pol1 = jnp.sum(jnp.maximum(_M - h1 * t, 0.0))

    neg = jnp.sum((h0 < 0).astype(jnp.float32)) + jnp.sum((h1 < 0).astype(jnp.float32))
    pos = jnp.sum((h0 > 0).astype(jnp.float32)) + jnp.sum((h1 > 0).astype(jnp.float32))

    acc_ref[0] += ce0
    acc_ref[1] += ce1
    acc_ref[2] += pol0
    acc_ref[3] += pol1
    acc_ref[4] += neg
    acc_ref[5] += pos

    @pl.when(i == _STEPS - 1)
    def _finish():
        cls_loss = 0.5 * (acc_ref[0] / _B) + 0.5 * (acc_ref[1] / _B)
        pol = (acc_ref[2] + acc_ref[3]) / (_B * _HB)
        denom = 2.0 * (2 * _HB) * _B
        p_m1 = acc_ref[4] / denom
        p_1 = acc_ref[5] / denom
        inv_ln2 = 1.4426950408889634
        b_loss = jnp.abs(-p_m1 * jnp.log(p_m1) * inv_ln2
                         + p_1 * jnp.log(p_1) * inv_ln2)
        out_ref[0] = cls_loss + _ALPHA * pol + _BETA * b_loss


def kernel(hash_out, cls_out, target, ind, target_vectors, U, Y):
    out = pl.pallas_call(
        _loss_kernel,
        grid=(_STEPS,),
        in_specs=[
            pl.BlockSpec((2, _CB, _HB), lambda i: (0, i, 0)),
            pl.BlockSpec((2, _CB, _NC), lambda i: (0, i, 0)),
            pl.BlockSpec((_CB, _NC), lambda i: (i, 0)),
            pl.BlockSpec((_NC, _HB), lambda i: (0, 0)),
        ],
        out_shape=jax.ShapeDtypeStruct((1,), jnp.float32),
        out_specs=pl.BlockSpec(memory_space=pltpu.SMEM),
        scratch_shapes=[pltpu.SMEM((8,), jnp.float32)],
    )(hash_out, cls_out, target, target_vectors)
    return out[0]


# speedup vs baseline: 2.0410x; 2.0410x over previous
import jax
import jax.numpy as jnp
from jax.experimental import pallas as pl
from jax.experimental.pallas import tpu as pltpu

_B = 4096
_NC = 100
_HB = 64
_M = 16.0
_ALPHA = 0.1
_BETA = 0.1


def _loss_kernel(hash_ref, cls_ref, target_ref, tv_ref, out_ref):
    tgt = target_ref[...]
    row = jax.lax.broadcasted_iota(jnp.int32, (_NC, _B), 0)
    cmax = jnp.max(tgt, axis=0, keepdims=True)
    label = jnp.min(jnp.where(tgt == cmax, row, _NC), axis=0, keepdims=True)
    onehot = (row == label).astype(jnp.float32)

    def ce_sum(logits):
        m = jnp.max(logits, axis=0, keepdims=True)
        lse = m + jnp.log(jnp.sum(jnp.exp(logits - m), axis=0, keepdims=True))
        picked = jnp.sum(onehot * logits, axis=0, keepdims=True)
        return jnp.sum(lse - picked)

    ce0 = ce_sum(cls_ref[0])
    ce1 = ce_sum(cls_ref[1])

    t = jnp.dot(tv_ref[...].astype(jnp.bfloat16), onehot.astype(jnp.bfloat16),
                preferred_element_type=jnp.float32)

    h0 = hash_ref[0]
    h1 = hash_ref[1]
    pol0 = jnp.sum(jnp.maximum(_M - h0 * t, 0.0))
    pol1 = jnp.sum(jnp.maximum(_M - h1 * t, 0.0))

    neg = jnp.sum((h0 < 0).astype(jnp.float32)) + jnp.sum((h1 < 0).astype(jnp.float32))
    pos = jnp.sum((h0 > 0).astype(jnp.float32)) + jnp.sum((h1 > 0).astype(jnp.float32))

    cls_loss = 0.5 * (ce0 / _B) + 0.5 * (ce1 / _B)
    pol = (pol0 + pol1) / (_B * _HB)
    denom = 2.0 * (2 * _HB) * _B
    p_m1 = neg / denom
    p_1 = pos / denom
    inv_ln2 = 1.4426950408889634
    b_loss = jnp.abs(-p_m1 * jnp.log(p_m1) * inv_ln2 + p_1 * jnp.log(p_1) * inv_ln2)
    out_ref[0] = cls_loss + _ALPHA * pol + _BETA * b_loss


def kernel(hash_out, cls_out, target, ind, target_vectors, U, Y):
    hT = jnp.transpose(hash_out, (0, 2, 1))
    cT = jnp.transpose(cls_out, (0, 2, 1))
    tT = target.T
    tvT = target_vectors.T
    out = pl.pallas_call(
        _loss_kernel,
        out_shape=jax.ShapeDtypeStruct((1,), jnp.float32),
        out_specs=pl.BlockSpec(memory_space=pltpu.SMEM),
    )(hT, cT, tT, tvT)
    return out[0]


# speedup vs baseline: 3.8342x; 1.8786x over previous
import jax
import jax.numpy as jnp
from jax.experimental import pallas as pl
from jax.experimental.pallas import tpu as pltpu

_B = 4096
_NC = 100
_HB = 64
_M = 16.0
_ALPHA = 0.1
_BETA = 0.1


def _loss_kernel(hash_ref, cls_ref, target_ref, tv_ref, out_ref):
    tgt = target_ref[...]
    row = jax.lax.broadcasted_iota(jnp.int32, (_NC, _B), 0)
    cmax = jnp.max(tgt, axis=0, keepdims=True)
    label = jnp.min(jnp.where(tgt == cmax, row, _NC), axis=0, keepdims=True)
    onehot = (row == label).astype(jnp.float32)

    label32 = label.reshape(32, 128)
    sel = (jax.lax.broadcasted_iota(jnp.int32, (64, 32), 1)
           == jax.lax.broadcasted_iota(jnp.int32, (64, 32), 0) // 2)
    label64 = jnp.dot(sel.astype(jnp.bfloat16),
                      label32.astype(jnp.bfloat16),
                      preferred_element_type=jnp.float32).astype(jnp.int32)

    cls3 = cls_ref[...]
    m3 = jnp.max(cls3, axis=0)
    s3 = jnp.sum(jnp.exp(cls3 - m3[None]), axis=0)
    lse3 = m3 + jnp.log(s3)
    c3 = jax.lax.broadcasted_iota(jnp.int32, (_NC, 64, 128), 0)
    picked3 = jnp.sum(jnp.where(c3 == label64[None], cls3, 0.0), axis=0)
    diff = lse3 - picked3
    par = jax.lax.broadcasted_iota(jnp.int32, (64, 128), 0) % 2
    ce0 = jnp.sum(jnp.where(par == 0, diff, 0.0))
    ce1 = jnp.sum(jnp.where(par == 1, diff, 0.0))

    t = jnp.dot(tv_ref[...].astype(jnp.bfloat16), onehot.astype(jnp.bfloat16),
                preferred_element_type=jnp.float32)

    h0 = hash_ref[0]
    h1 = hash_ref[1]
    pol0 = jnp.sum(jnp.maximum(_M - h0 * t, 0.0))
    pol1 = jnp.sum(jnp.maximum(_M - h1 * t, 0.0))

    neg = jnp.sum((h0 < 0).astype(jnp.float32)) + jnp.sum((h1 < 0).astype(jnp.float32))
    pos = jnp.sum((h0 > 0).astype(jnp.float32)) + jnp.sum((h1 > 0).astype(jnp.float32))

    cls_loss = 0.5 * (ce0 / _B) + 0.5 * (ce1 / _B)
    pol = (pol0 + pol1) / (_B * _HB)
    denom = 2.0 * (2 * _HB) * _B
    p_m1 = neg / denom
    p_1 = pos / denom
    inv_ln2 = 1.4426950408889634
    b_loss = jnp.abs(-p_m1 * jnp.log(p_m1) * inv_ln2 + p_1 * jnp.log(p_1) * inv_ln2)
    out_ref[0] = cls_loss + _ALPHA * pol + _BETA * b_loss


def kernel(hash_out, cls_out, target, ind, target_vectors, U, Y):
    hT = jnp.transpose(hash_out, (0, 2, 1))
    tT = target.T
    tvT = target_vectors.T
    cls3 = (cls_out.transpose(2, 1, 0)
            .reshape(_NC, 32, 128, 2)
            .transpose(0, 1, 3, 2)
            .reshape(_NC, 64, 128))
    out = pl.pallas_call(
        _loss_kernel,
        out_shape=jax.ShapeDtypeStruct((1,), jnp.float32),
        out_specs=pl.BlockSpec(memory_space=pltpu.SMEM),
    )(hT, cls3, tT, tvT)
    return out[0]
